# baseline (speedup 1.0000x reference)
import functools

import jax
import jax.numpy as jnp
from jax import lax
from jax.experimental import pallas as pl
from jax.experimental.pallas import tpu as pltpu
from jax.experimental.pallas import tpu_sc as plsc

_D = 64
_BATCH = 4096
_SEQ = 200
_B_TOTAL = _BATCH * _SEQ
_NC = 2
_NS = 16
_NW = _NC * _NS
_B_PER_W = _B_TOTAL // _NW
_CHUNK = 1600
_N_ITERS = _B_PER_W // _CHUNK


def _gather_body(table_hbm, idx_hbm, out_hbm, idx_v, rows_v, sem):
    wid = lax.axis_index("s") * _NC + lax.axis_index("c")
    base0 = wid * _B_PER_W

    def body(i, carry):
        base = base0 + i * _CHUNK
        pltpu.sync_copy(idx_hbm.at[pl.ds(base, _CHUNK)], idx_v)
        pltpu.async_copy(table_hbm.at[idx_v], rows_v, sem).wait()
        pltpu.sync_copy(rows_v, out_hbm.at[pl.ds(base, _CHUNK)])
        return carry

    lax.fori_loop(0, _N_ITERS, body, 0)


@jax.jit
def kernel(x, vocab_table):
    mesh = plsc.VectorSubcoreMesh(core_axis_name="c", subcore_axis_name="s")
    gather = functools.partial(
        pl.kernel,
        mesh=mesh,
        out_type=jax.ShapeDtypeStruct((_B_TOTAL, _D), jnp.float32),
        scratch_types=[
            pltpu.VMEM((_CHUNK,), jnp.int32),
            pltpu.VMEM((_CHUNK, _D), jnp.float32),
            pltpu.SemaphoreType.DMA,
        ],
        compiler_params=pltpu.CompilerParams(use_tc_tiling_on_sc=False),
    )(_gather_body)
    out = gather(vocab_table, x.reshape(_B_TOTAL))
    return out.reshape(_BATCH, _SEQ, _D)


# speedup vs baseline: 1.0013x; 1.0013x over previous
import functools

import jax
import jax.numpy as jnp
from jax import lax
from jax.experimental import pallas as pl
from jax.experimental.pallas import tpu as pltpu
from jax.experimental.pallas import tpu_sc as plsc

_D = 64
_BATCH = 4096
_SEQ = 200
_B_TOTAL = _BATCH * _SEQ
_NC = 2
_NS = 16
_NW = _NC * _NS
_B_PER_W = _B_TOTAL // _NW
_CHUNK = 800
_N_ITERS = _B_PER_W // _CHUNK
_NBUF = 2
_N_GROUPS = _N_ITERS // _NBUF


def _gather_body(table_hbm, idx_hbm, out_hbm, idx_v, buf0, buf1,
                 sg0, sg1, sw0, sw1):
    wid = lax.axis_index("s") * _NC + lax.axis_index("c")
    base0 = wid * _B_PER_W
    bufs = (buf0, buf1)
    sgs = (sg0, sg1)
    sws = (sw0, sw1)

    def start_gather(i, b):
        pltpu.async_copy(table_hbm.at[idx_v.at[i]], bufs[b], sgs[b])

    def wait_gather(b):
        pltpu.make_async_copy(table_hbm.at[idx_v.at[0]], bufs[b], sgs[b]).wait()

    def start_write(i, b):
        dst = out_hbm.at[pl.ds(base0 + i * _CHUNK, _CHUNK)]
        pltpu.async_copy(bufs[b], dst, sws[b])

    def wait_write(b):
        dst = out_hbm.at[pl.ds(base0, _CHUNK)]
        pltpu.make_async_copy(bufs[b], dst, sws[b]).wait()

    pltpu.sync_copy(idx_hbm.at[wid], idx_v)

    start_gather(0, 0)
    start_gather(1, 1)

    def group(g, carry):
        for b in range(_NBUF):
            i = g * _NBUF + b
            wait_gather(b)
            start_write(i, b)

        @pl.when(g < _N_GROUPS - 1)
        def _():
            for b in range(_NBUF):
                i = g * _NBUF + b
                wait_write(b)
                start_gather(i + _NBUF, b)

        return carry

    lax.fori_loop(0, _N_GROUPS, group, 0)

    for b in range(_NBUF):
        wait_write(b)


@jax.jit
def kernel(x, vocab_table):
    mesh = plsc.VectorSubcoreMesh(core_axis_name="c", subcore_axis_name="s")
    gather = functools.partial(
        pl.kernel,
        mesh=mesh,
        out_type=jax.ShapeDtypeStruct((_B_TOTAL, _D), jnp.float32),
        scratch_types=[
            pltpu.VMEM((_N_ITERS, _CHUNK), jnp.int32),
            pltpu.VMEM((_CHUNK, _D), jnp.float32),
            pltpu.VMEM((_CHUNK, _D), jnp.float32),
            pltpu.SemaphoreType.DMA,
            pltpu.SemaphoreType.DMA,
            pltpu.SemaphoreType.DMA,
            pltpu.SemaphoreType.DMA,
        ],
        compiler_params=pltpu.CompilerParams(use_tc_tiling_on_sc=False),
    )(_gather_body)
    out = gather(vocab_table, x.reshape(_NW, _N_ITERS, _CHUNK))
    return out.reshape(_BATCH, _SEQ, _D)
